# baseline (speedup 1.0000x reference)
import functools

import jax
import jax.numpy as jnp
from jax import lax
from jax.experimental import pallas as pl
from jax.experimental.pallas import tpu as pltpu
from jax.experimental.pallas import tpu_sc as plsc

_T = 2048
_E = 64
_D = 768
_BT = 32
_N_ROWS = 2528
_NW = 32
_TPW = _T // _NW

_MESH = dict(core_axis_name="c", subcore_axis_name="s")


def _route_dispatch_body(idx_hbm, w_hbm, x_hbm,
                         poff_hbm, cnt_hbm, pos_hbm, wpad_hbm, xpad_hbm,
                         idx_t, pos_t, hist_v, lbase_v, cntr_v, thist_v,
                         ghist_v, tbase_v, poff_v, cnts_v,
                         pos_all_v, w_all_v, wpad_v, pos_d, rows_v,
                         hist_sh, pos_sh, sem):
    c = lax.axis_index("c")
    s = lax.axis_index("s")
    lanes = lax.iota(jnp.int32, 16)
    ones = jnp.ones((16,), jnp.int32)
    zeros = jnp.zeros((16,), jnp.int32)

    pltpu.sync_copy(idx_hbm.at[pl.ds(s * 128, 128)], idx_t)

    def zbody(i, carry):
        hist_v[pl.ds(i * 16, 16)] = zeros
        cntr_v[pl.ds(i * 16, 16)] = zeros
        return carry

    lax.fori_loop(0, 64, zbody, 0)

    for j in range(8):
        e = idx_t[pl.ds(j * 16, 16)]
        plsc.addupdate_scatter(hist_v, [lanes * 64 + e], ones)

    for ch in range(4):
        run = zeros
        for l in range(16):
            lbase_v[pl.ds(l * 64 + ch * 16, 16)] = run
            run = run + hist_v[pl.ds(l * 64 + ch * 16, 16)]
        thist_v[pl.ds(ch * 16, 16)] = run

    pltpu.sync_copy(thist_v, hist_sh.at[pl.ds(s * 64, 64)])
    plsc.subcore_barrier()
    pltpu.sync_copy(hist_sh, ghist_v)
    carry = zeros
    for ch in range(4):
        tbase = zeros
        total = zeros
        for t in range(16):
            g = ghist_v[pl.ds(t * 64 + ch * 16, 16)]
            tbase = tbase + jnp.where(jnp.broadcast_to(t < s, (16,)), g, zeros)
            total = total + g
        tbase_v[pl.ds(ch * 16, 16)] = tbase
        cnts_v[pl.ds(ch * 16, 16)] = total
        pc = ((total + 7) >> 3) << 3
        incl = plsc.cumsum(pc)
        poff_v[pl.ds(ch * 16, 16)] = incl - pc + carry
        carry = carry + jnp.broadcast_to(jnp.sum(pc), (16,))

    for j in range(8):
        e = idx_t[pl.ds(j * 16, 16)]
        stripe = lanes * 64 + e
        base = plsc.load_gather(poff_v, [e])
        tb = plsc.load_gather(tbase_v, [e])
        lb = plsc.load_gather(lbase_v, [stripe])
        cr = plsc.load_gather(cntr_v, [stripe])
        p = base + tb + lb + cr
        plsc.store_scatter(cntr_v, [stripe], cr + ones)
        pos_t[pl.ds(j * 16, 16)] = p

    pltpu.sync_copy(pos_t, pos_sh.at[pl.ds(s * 128, 128)])

    @pl.when(c == 0)
    def _():
        pltpu.sync_copy(pos_t, pos_hbm.at[pl.ds(s * 128, 128)])

    @pl.when((c == 0) & (s == 0))
    def _():
        pltpu.sync_copy(poff_v, poff_hbm)
        pltpu.sync_copy(cnts_v, cnt_hbm)

    plsc.subcore_barrier()

    @pl.when((c == 0) & (s == 0))
    def _():
        pltpu.sync_copy(pos_sh, pos_all_v)
        pltpu.sync_copy(w_hbm, w_all_v)

        def wbody(j, carry):
            wv = w_all_v[pl.ds(j * 16, 16)]
            p = pos_all_v[pl.ds(j * 16, 16)]
            plsc.store_scatter(wpad_v, [p], wv)
            return carry

        lax.fori_loop(0, 128, wbody, 0)
        pltpu.sync_copy(wpad_v, wpad_hbm)

    wid = s * 2 + c
    dbase = wid * _TPW
    pltpu.sync_copy(pos_sh.at[pl.ds(dbase, _TPW)], pos_d)
    pltpu.sync_copy(x_hbm.at[pl.ds(dbase, _TPW)], rows_v)
    pltpu.async_copy(rows_v, xpad_hbm.at[pos_d], sem).wait()


def _route_dispatch(idx, w, x):
    return pl.kernel(
        _route_dispatch_body,
        out_type=[
            jax.ShapeDtypeStruct((_E,), jnp.int32),
            jax.ShapeDtypeStruct((_E,), jnp.int32),
            jax.ShapeDtypeStruct((_T,), jnp.int32),
            jax.ShapeDtypeStruct((_N_ROWS,), jnp.float32),
            jax.ShapeDtypeStruct((_N_ROWS, _D), jnp.float32),
        ],
        mesh=plsc.VectorSubcoreMesh(**_MESH),
        compiler_params=pltpu.CompilerParams(needs_layout_passes=False),
        scratch_types=[
            pltpu.VMEM((128,), jnp.int32),
            pltpu.VMEM((128,), jnp.int32),
            pltpu.VMEM((1024,), jnp.int32),
            pltpu.VMEM((1024,), jnp.int32),
            pltpu.VMEM((1024,), jnp.int32),
            pltpu.VMEM((_E,), jnp.int32),
            pltpu.VMEM((1024,), jnp.int32),
            pltpu.VMEM((_E,), jnp.int32),
            pltpu.VMEM((_E,), jnp.int32),
            pltpu.VMEM((_E,), jnp.int32),
            pltpu.VMEM((_T,), jnp.int32),
            pltpu.VMEM((_T,), jnp.float32),
            pltpu.VMEM((_N_ROWS,), jnp.float32),
            pltpu.VMEM((_TPW,), jnp.int32),
            pltpu.VMEM((_TPW, _D), jnp.float32),
            pltpu.VMEM_SHARED((1024,), jnp.int32),
            pltpu.VMEM_SHARED((_T,), jnp.int32),
            pltpu.SemaphoreType.DMA,
        ],
    )(idx, w, x)


def _gmm_body(poff_ref, cnt_ref, x_ref, w_ref, fc1_ref, fc2_ref, out_ref, *,
              bt, d_half, epb):
    g = pl.program_id(0)
    for k in range(epb):
        e = g * epb + k
        start_e = poff_ref[e]
        nt = (cnt_ref[e] + bt - 1) // bt

        def body(i, carry, k=k, start_e=start_e):
            start = pl.multiple_of(start_e + i * bt, 8)
            rows = x_ref[pl.ds(start, bt), :]
            y = lax.dot_general(rows, fc1_ref[k], (((1,), (1,)), ((), ())),
                                preferred_element_type=jnp.float32)
            y1 = y[:, :d_half]
            gate = y[:, d_half:]
            h = y1 * (gate * jax.nn.sigmoid(gate))
            yo = lax.dot_general(h, fc2_ref[k], (((1,), (1,)), ((), ())),
                                 preferred_element_type=jnp.float32)
            out_ref[pl.ds(start, bt), :] = yo * w_ref[pl.ds(start, bt), :]
            return carry

        lax.fori_loop(0, nt, body, 0)


def _grouped_mlp(poff, counts, x_pad, w_pad, fc1_weights, fc2_weights, *,
                 bt=_BT, epb=4):
    n_rows, d_model = x_pad.shape
    n_experts, d_ff2, _ = fc1_weights.shape
    d_half = d_ff2 // 2
    grid_spec = pltpu.PrefetchScalarGridSpec(
        num_scalar_prefetch=2,
        grid=(n_experts // epb,),
        in_specs=[
            pl.BlockSpec((n_rows, d_model), lambda g, poff, cnt: (0, 0)),
            pl.BlockSpec((n_rows, 1), lambda g, poff, cnt: (0, 0)),
            pl.BlockSpec((epb, d_ff2, d_model), lambda g, poff, cnt: (g, 0, 0)),
            pl.BlockSpec((epb, d_model, d_half), lambda g, poff, cnt: (g, 0, 0)),
        ],
        out_specs=pl.BlockSpec((n_rows, d_model), lambda g, poff, cnt: (0, 0)),
    )
    return pl.pallas_call(
        functools.partial(_gmm_body, bt=bt, d_half=d_half, epb=epb),
        grid_spec=grid_spec,
        out_shape=jax.ShapeDtypeStruct((n_rows, d_model), jnp.float32),
    )(poff, counts, x_pad, w_pad, fc1_weights, fc2_weights)


def _combine_body(outpad_hbm, pos_hbm, out_hbm, pos_v, rows_v, sem):
    wid = lax.axis_index("s") * 2 + lax.axis_index("c")
    base = wid * _TPW
    pltpu.sync_copy(pos_hbm.at[pl.ds(base, _TPW)], pos_v)
    pltpu.async_copy(outpad_hbm.at[pos_v], rows_v, sem).wait()
    pltpu.sync_copy(rows_v, out_hbm.at[pl.ds(base, _TPW)])


def _combine(out_pad, pos):
    return pl.kernel(
        _combine_body,
        out_type=jax.ShapeDtypeStruct((_T, _D), jnp.float32),
        mesh=plsc.VectorSubcoreMesh(**_MESH),
        scratch_types=[
            pltpu.VMEM((_TPW,), jnp.int32),
            pltpu.VMEM((_TPW, _D), jnp.float32),
            pltpu.SemaphoreType.DMA,
        ],
    )(out_pad, pos)


def kernel(x, weights, indices, fc1_weights, fc2_weights):
    idx = indices.reshape(-1).astype(jnp.int32)
    w = weights.reshape(-1)
    poff, counts, pos, w_pad, x_pad = _route_dispatch(idx, w, x)
    out_pad = _grouped_mlp(poff, counts, x_pad, w_pad.reshape(-1, 1),
                           fc1_weights, fc2_weights)
    return _combine(out_pad, pos)


# speedup vs baseline: 1.0800x; 1.0800x over previous
import functools

import jax
import jax.numpy as jnp
from jax import lax
from jax.experimental import pallas as pl
from jax.experimental.pallas import tpu as pltpu
from jax.experimental.pallas import tpu_sc as plsc

_T = 2048
_E = 64
_D = 768
_BT = 64
_N_ROWS = 2560
_NW = 32
_TPW = _T // _NW

_MESH = dict(core_axis_name="c", subcore_axis_name="s")


def _route_dispatch_body(idx_hbm, w_hbm, x_hbm,
                         poff_hbm, cnt_hbm, pos_hbm, wpad_hbm, xpad_hbm,
                         idx_t, pos_t, hist_v, lbase_v, cntr_v, thist_v,
                         ghist_v, tbase_v, poff_v, cnts_v,
                         pos_all_v, w_all_v, wpad_v, pos_d, rows_v,
                         hist_sh, pos_sh, sem):
    c = lax.axis_index("c")
    s = lax.axis_index("s")
    lanes = lax.iota(jnp.int32, 16)
    ones = jnp.ones((16,), jnp.int32)
    zeros = jnp.zeros((16,), jnp.int32)

    pltpu.sync_copy(idx_hbm.at[pl.ds(s * 128, 128)], idx_t)

    def zbody(i, carry):
        hist_v[pl.ds(i * 16, 16)] = zeros
        cntr_v[pl.ds(i * 16, 16)] = zeros
        return carry

    lax.fori_loop(0, 64, zbody, 0)

    for j in range(8):
        e = idx_t[pl.ds(j * 16, 16)]
        plsc.addupdate_scatter(hist_v, [lanes * 64 + e], ones)

    for ch in range(4):
        run = zeros
        for l in range(16):
            lbase_v[pl.ds(l * 64 + ch * 16, 16)] = run
            run = run + hist_v[pl.ds(l * 64 + ch * 16, 16)]
        thist_v[pl.ds(ch * 16, 16)] = run

    pltpu.sync_copy(thist_v, hist_sh.at[pl.ds(s * 64, 64)])
    plsc.subcore_barrier()
    pltpu.sync_copy(hist_sh, ghist_v)
    carry = zeros
    for ch in range(4):
        tbase = zeros
        total = zeros
        for t in range(16):
            g = ghist_v[pl.ds(t * 64 + ch * 16, 16)]
            tbase = tbase + jnp.where(jnp.broadcast_to(t < s, (16,)), g, zeros)
            total = total + g
        tbase_v[pl.ds(ch * 16, 16)] = tbase
        cnts_v[pl.ds(ch * 16, 16)] = total
        pc = ((total + 7) >> 3) << 3
        incl = plsc.cumsum(pc)
        poff_v[pl.ds(ch * 16, 16)] = incl - pc + carry
        carry = carry + jnp.broadcast_to(jnp.sum(pc), (16,))

    for j in range(8):
        e = idx_t[pl.ds(j * 16, 16)]
        stripe = lanes * 64 + e
        base = plsc.load_gather(poff_v, [e])
        tb = plsc.load_gather(tbase_v, [e])
        lb = plsc.load_gather(lbase_v, [stripe])
        cr = plsc.load_gather(cntr_v, [stripe])
        p = base + tb + lb + cr
        plsc.store_scatter(cntr_v, [stripe], cr + ones)
        pos_t[pl.ds(j * 16, 16)] = p

    pltpu.sync_copy(pos_t, pos_sh.at[pl.ds(s * 128, 128)])

    @pl.when(c == 0)
    def _():
        pltpu.sync_copy(pos_t, pos_hbm.at[pl.ds(s * 128, 128)])

    @pl.when((c == 0) & (s == 0))
    def _():
        pltpu.sync_copy(poff_v, poff_hbm)
        pltpu.sync_copy(cnts_v, cnt_hbm)

    plsc.subcore_barrier()

    @pl.when((c == 0) & (s == 0))
    def _():
        pltpu.sync_copy(pos_sh, pos_all_v)
        pltpu.sync_copy(w_hbm, w_all_v)

        def wbody(j, carry):
            wv = w_all_v[pl.ds(j * 16, 16)]
            p = pos_all_v[pl.ds(j * 16, 16)]
            plsc.store_scatter(wpad_v, [p], wv)
            return carry

        lax.fori_loop(0, 128, wbody, 0)
        pltpu.sync_copy(wpad_v, wpad_hbm)

    wid = s * 2 + c
    dbase = wid * _TPW
    pltpu.sync_copy(pos_sh.at[pl.ds(dbase, _TPW)], pos_d)
    pltpu.sync_copy(x_hbm.at[pl.ds(dbase, _TPW)], rows_v)
    pltpu.async_copy(rows_v, xpad_hbm.at[pos_d], sem).wait()


def _route_dispatch(idx, w, x):
    return pl.kernel(
        _route_dispatch_body,
        out_type=[
            jax.ShapeDtypeStruct((_E,), jnp.int32),
            jax.ShapeDtypeStruct((_E,), jnp.int32),
            jax.ShapeDtypeStruct((_T,), jnp.int32),
            jax.ShapeDtypeStruct((_N_ROWS,), jnp.float32),
            jax.ShapeDtypeStruct((_N_ROWS, _D), jnp.float32),
        ],
        mesh=plsc.VectorSubcoreMesh(**_MESH),
        compiler_params=pltpu.CompilerParams(needs_layout_passes=False),
        scratch_types=[
            pltpu.VMEM((128,), jnp.int32),
            pltpu.VMEM((128,), jnp.int32),
            pltpu.VMEM((1024,), jnp.int32),
            pltpu.VMEM((1024,), jnp.int32),
            pltpu.VMEM((1024,), jnp.int32),
            pltpu.VMEM((_E,), jnp.int32),
            pltpu.VMEM((1024,), jnp.int32),
            pltpu.VMEM((_E,), jnp.int32),
            pltpu.VMEM((_E,), jnp.int32),
            pltpu.VMEM((_E,), jnp.int32),
            pltpu.VMEM((_T,), jnp.int32),
            pltpu.VMEM((_T,), jnp.float32),
            pltpu.VMEM((_N_ROWS,), jnp.float32),
            pltpu.VMEM((_TPW,), jnp.int32),
            pltpu.VMEM((_TPW, _D), jnp.float32),
            pltpu.VMEM_SHARED((1024,), jnp.int32),
            pltpu.VMEM_SHARED((_T,), jnp.int32),
            pltpu.SemaphoreType.DMA,
        ],
    )(idx, w, x)


def _gmm_body(poff_ref, cnt_ref, x_ref, w_ref, fc1_ref, fc2_ref, out_ref, *,
              bt, d_half, epb):
    g = pl.program_id(0)
    for k in range(epb):
        e = g * epb + k
        start_e = poff_ref[e]
        nt = (cnt_ref[e] + bt - 1) // bt

        def body(i, carry, k=k, start_e=start_e):
            start = pl.multiple_of(start_e + i * bt, 8)
            rows = x_ref[pl.ds(start, bt), :]
            y = lax.dot_general(rows, fc1_ref[k], (((1,), (1,)), ((), ())),
                                preferred_element_type=jnp.float32)
            y1 = y[:, :d_half]
            gate = y[:, d_half:]
            h = y1 * (gate * jax.nn.sigmoid(gate))
            yo = lax.dot_general(h, fc2_ref[k], (((1,), (1,)), ((), ())),
                                 preferred_element_type=jnp.float32)
            out_ref[pl.ds(start, bt), :] = yo * w_ref[pl.ds(start, bt), :]
            return carry

        lax.fori_loop(0, nt, body, 0)


def _grouped_mlp(poff, counts, x_pad, w_pad, fc1_weights, fc2_weights, *,
                 bt=_BT, epb=4):
    n_rows, d_model = x_pad.shape
    n_experts, d_ff2, _ = fc1_weights.shape
    d_half = d_ff2 // 2
    grid_spec = pltpu.PrefetchScalarGridSpec(
        num_scalar_prefetch=2,
        grid=(n_experts // epb,),
        in_specs=[
            pl.BlockSpec((n_rows, d_model), lambda g, poff, cnt: (0, 0)),
            pl.BlockSpec((n_rows, 1), lambda g, poff, cnt: (0, 0)),
            pl.BlockSpec((epb, d_ff2, d_model), lambda g, poff, cnt: (g, 0, 0)),
            pl.BlockSpec((epb, d_model, d_half), lambda g, poff, cnt: (g, 0, 0)),
        ],
        out_specs=pl.BlockSpec((n_rows, d_model), lambda g, poff, cnt: (0, 0)),
    )
    return pl.pallas_call(
        functools.partial(_gmm_body, bt=bt, d_half=d_half, epb=epb),
        grid_spec=grid_spec,
        out_shape=jax.ShapeDtypeStruct((n_rows, d_model), jnp.float32),
    )(poff, counts, x_pad, w_pad, fc1_weights, fc2_weights)


def _combine_body(outpad_hbm, pos_hbm, out_hbm, pos_v, rows_v, sem):
    wid = lax.axis_index("s") * 2 + lax.axis_index("c")
    base = wid * _TPW
    pltpu.sync_copy(pos_hbm.at[pl.ds(base, _TPW)], pos_v)
    pltpu.async_copy(outpad_hbm.at[pos_v], rows_v, sem).wait()
    pltpu.sync_copy(rows_v, out_hbm.at[pl.ds(base, _TPW)])


def _combine(out_pad, pos):
    return pl.kernel(
        _combine_body,
        out_type=jax.ShapeDtypeStruct((_T, _D), jnp.float32),
        mesh=plsc.VectorSubcoreMesh(**_MESH),
        scratch_types=[
            pltpu.VMEM((_TPW,), jnp.int32),
            pltpu.VMEM((_TPW, _D), jnp.float32),
            pltpu.SemaphoreType.DMA,
        ],
    )(out_pad, pos)


def kernel(x, weights, indices, fc1_weights, fc2_weights):
    idx = indices.reshape(-1).astype(jnp.int32)
    w = weights.reshape(-1)
    poff, counts, pos, w_pad, x_pad = _route_dispatch(idx, w, x)
    out_pad = _grouped_mlp(poff, counts, x_pad, w_pad.reshape(-1, 1),
                           fc1_weights, fc2_weights)
    return _combine(out_pad, pos)


# speedup vs baseline: 1.1081x; 1.0260x over previous
import functools

import jax
import jax.numpy as jnp
from jax import lax
from jax.experimental import pallas as pl
from jax.experimental.pallas import tpu as pltpu
from jax.experimental.pallas import tpu_sc as plsc

_T = 2048
_E = 64
_D = 768
_BT = 64
_N_ROWS = 2560
_NW = 32
_TPW = _T // _NW

_MESH = dict(core_axis_name="c", subcore_axis_name="s")


def _route_dispatch_body(idx_hbm, w_hbm, x_hbm,
                         poff_hbm, cnt_hbm, pos_hbm, wpad_hbm, xpad_hbm,
                         idx_t, pos_t, hist_v, lbase_v, cntr_v, thist_v,
                         ghist_v, tbase_v, poff_v, cnts_v,
                         pos_all_v, w_all_v, wpad_v, pos_d, rows_v,
                         hist_sh, pos_sh, sem):
    c = lax.axis_index("c")
    s = lax.axis_index("s")
    lanes = lax.iota(jnp.int32, 16)
    ones = jnp.ones((16,), jnp.int32)
    zeros = jnp.zeros((16,), jnp.int32)

    pltpu.sync_copy(idx_hbm.at[pl.ds(s * 128, 128)], idx_t)

    wid = s * 2 + c
    dbase = wid * _TPW
    xcp = pltpu.async_copy(x_hbm.at[pl.ds(dbase, _TPW)], rows_v, sem)

    def zbody(i, carry):
        hist_v[pl.ds(i * 16, 16)] = zeros
        cntr_v[pl.ds(i * 16, 16)] = zeros
        return carry

    lax.fori_loop(0, 64, zbody, 0)

    for j in range(8):
        e = idx_t[pl.ds(j * 16, 16)]
        plsc.addupdate_scatter(hist_v, [lanes * 64 + e], ones)

    for ch in range(4):
        run = zeros
        for l in range(16):
            lbase_v[pl.ds(l * 64 + ch * 16, 16)] = run
            run = run + hist_v[pl.ds(l * 64 + ch * 16, 16)]
        thist_v[pl.ds(ch * 16, 16)] = run

    pltpu.sync_copy(thist_v, hist_sh.at[pl.ds(s * 64, 64)])
    plsc.subcore_barrier()
    pltpu.sync_copy(hist_sh, ghist_v)
    carry = zeros
    for ch in range(4):
        tbase = zeros
        total = zeros
        for t in range(16):
            g = ghist_v[pl.ds(t * 64 + ch * 16, 16)]
            tbase = tbase + jnp.where(jnp.broadcast_to(t < s, (16,)), g, zeros)
            total = total + g
        tbase_v[pl.ds(ch * 16, 16)] = tbase
        cnts_v[pl.ds(ch * 16, 16)] = total
        pc = ((total + 7) >> 3) << 3
        incl = plsc.cumsum(pc)
        poff_v[pl.ds(ch * 16, 16)] = incl - pc + carry
        carry = carry + jnp.broadcast_to(jnp.sum(pc), (16,))

    for j in range(8):
        e = idx_t[pl.ds(j * 16, 16)]
        stripe = lanes * 64 + e
        base = plsc.load_gather(poff_v, [e])
        tb = plsc.load_gather(tbase_v, [e])
        lb = plsc.load_gather(lbase_v, [stripe])
        cr = plsc.load_gather(cntr_v, [stripe])
        p = base + tb + lb + cr
        plsc.store_scatter(cntr_v, [stripe], cr + ones)
        pos_t[pl.ds(j * 16, 16)] = p

    pltpu.sync_copy(pos_t, pos_sh.at[pl.ds(s * 128, 128)])

    @pl.when(c == 0)
    def _():
        pltpu.sync_copy(pos_t, pos_hbm.at[pl.ds(s * 128, 128)])

    @pl.when((c == 0) & (s == 0))
    def _():
        pltpu.sync_copy(poff_v, poff_hbm)
        pltpu.sync_copy(cnts_v, cnt_hbm)

    plsc.subcore_barrier()

    @pl.when((c == 0) & (s == 0))
    def _():
        pltpu.sync_copy(pos_sh, pos_all_v)
        pltpu.sync_copy(w_hbm, w_all_v)

        def wbody(j, carry):
            wv = w_all_v[pl.ds(j * 16, 16)]
            p = pos_all_v[pl.ds(j * 16, 16)]
            plsc.store_scatter(wpad_v, [p], wv)
            return carry

        lax.fori_loop(0, 128, wbody, 0)
        pltpu.sync_copy(wpad_v, wpad_hbm)

    pltpu.sync_copy(pos_sh.at[pl.ds(dbase, _TPW)], pos_d)
    xcp.wait()
    pltpu.async_copy(rows_v, xpad_hbm.at[pos_d], sem).wait()


def _route_dispatch(idx, w, x):
    return pl.kernel(
        _route_dispatch_body,
        out_type=[
            jax.ShapeDtypeStruct((_E,), jnp.int32),
            jax.ShapeDtypeStruct((_E,), jnp.int32),
            jax.ShapeDtypeStruct((_T,), jnp.int32),
            jax.ShapeDtypeStruct((_N_ROWS,), jnp.float32),
            jax.ShapeDtypeStruct((_N_ROWS, _D), jnp.float32),
        ],
        mesh=plsc.VectorSubcoreMesh(**_MESH),
        compiler_params=pltpu.CompilerParams(needs_layout_passes=False),
        scratch_types=[
            pltpu.VMEM((128,), jnp.int32),
            pltpu.VMEM((128,), jnp.int32),
            pltpu.VMEM((1024,), jnp.int32),
            pltpu.VMEM((1024,), jnp.int32),
            pltpu.VMEM((1024,), jnp.int32),
            pltpu.VMEM((_E,), jnp.int32),
            pltpu.VMEM((1024,), jnp.int32),
            pltpu.VMEM((_E,), jnp.int32),
            pltpu.VMEM((_E,), jnp.int32),
            pltpu.VMEM((_E,), jnp.int32),
            pltpu.VMEM((_T,), jnp.int32),
            pltpu.VMEM((_T,), jnp.float32),
            pltpu.VMEM((_N_ROWS,), jnp.float32),
            pltpu.VMEM((_TPW,), jnp.int32),
            pltpu.VMEM((_TPW, _D), jnp.float32),
            pltpu.VMEM_SHARED((1024,), jnp.int32),
            pltpu.VMEM_SHARED((_T,), jnp.int32),
            pltpu.SemaphoreType.DMA,
        ],
    )(idx, w, x)


def _gmm_body(poff_ref, cnt_ref, x_ref, w_ref, fc1_ref, fc2_ref, out_ref, *,
              bt, d_half, epb):
    g = pl.program_id(0)
    for k in range(epb):
        e = g * epb + k
        start_e = poff_ref[e]
        nt = (cnt_ref[e] + bt - 1) // bt

        def body(i, carry, k=k, start_e=start_e):
            start = pl.multiple_of(start_e + i * bt, 8)
            rows = x_ref[pl.ds(start, bt), :]
            y = lax.dot_general(rows, fc1_ref[k], (((1,), (1,)), ((), ())),
                                preferred_element_type=jnp.float32)
            y1 = y[:, :d_half]
            gate = y[:, d_half:]
            h = y1 * (gate * jax.nn.sigmoid(gate))
            yo = lax.dot_general(h, fc2_ref[k], (((1,), (1,)), ((), ())),
                                 preferred_element_type=jnp.float32)
            out_ref[pl.ds(start, bt), :] = yo * w_ref[pl.ds(start, bt), :]
            return carry

        lax.fori_loop(0, nt, body, 0)


def _grouped_mlp(poff, counts, x_pad, w_pad, fc1_weights, fc2_weights, *,
                 bt=_BT, epb=4):
    n_rows, d_model = x_pad.shape
    n_experts, d_ff2, _ = fc1_weights.shape
    d_half = d_ff2 // 2
    grid_spec = pltpu.PrefetchScalarGridSpec(
        num_scalar_prefetch=2,
        grid=(n_experts // epb,),
        in_specs=[
            pl.BlockSpec((n_rows, d_model), lambda g, poff, cnt: (0, 0)),
            pl.BlockSpec((n_rows, 1), lambda g, poff, cnt: (0, 0)),
            pl.BlockSpec((epb, d_ff2, d_model), lambda g, poff, cnt: (g, 0, 0)),
            pl.BlockSpec((epb, d_model, d_half), lambda g, poff, cnt: (g, 0, 0)),
        ],
        out_specs=pl.BlockSpec((n_rows, d_model), lambda g, poff, cnt: (0, 0)),
    )
    return pl.pallas_call(
        functools.partial(_gmm_body, bt=bt, d_half=d_half, epb=epb),
        grid_spec=grid_spec,
        out_shape=jax.ShapeDtypeStruct((n_rows, d_model), jnp.float32),
    )(poff, counts, x_pad, w_pad, fc1_weights, fc2_weights)


def _combine_body(outpad_hbm, pos_hbm, out_hbm, pos_v, rows_v, sem):
    wid = lax.axis_index("s") * 2 + lax.axis_index("c")
    base = wid * _TPW
    pltpu.sync_copy(pos_hbm.at[pl.ds(base, _TPW)], pos_v)
    pltpu.async_copy(outpad_hbm.at[pos_v], rows_v, sem).wait()
    pltpu.sync_copy(rows_v, out_hbm.at[pl.ds(base, _TPW)])


def _combine(out_pad, pos):
    return pl.kernel(
        _combine_body,
        out_type=jax.ShapeDtypeStruct((_T, _D), jnp.float32),
        mesh=plsc.VectorSubcoreMesh(**_MESH),
        scratch_types=[
            pltpu.VMEM((_TPW,), jnp.int32),
            pltpu.VMEM((_TPW, _D), jnp.float32),
            pltpu.SemaphoreType.DMA,
        ],
    )(out_pad, pos)


def kernel(x, weights, indices, fc1_weights, fc2_weights):
    idx = indices.reshape(-1).astype(jnp.int32)
    w = weights.reshape(-1)
    poff, counts, pos, w_pad, x_pad = _route_dispatch(idx, w, x)
    out_pad = _grouped_mlp(poff, counts, x_pad, w_pad.reshape(-1, 1),
                           fc1_weights, fc2_weights)
    return _combine(out_pad, pos)


# speedup vs baseline: 1.1086x; 1.0004x over previous
import functools

import jax
import jax.numpy as jnp
from jax import lax
from jax.experimental import pallas as pl
from jax.experimental.pallas import tpu as pltpu
from jax.experimental.pallas import tpu_sc as plsc

_T = 2048
_E = 64
_D = 768
_BT = 64
_N_ROWS = 2560
_NW = 32
_TPW = _T // _NW

_MESH = dict(core_axis_name="c", subcore_axis_name="s")


def _route_dispatch_body(idx_hbm, w_hbm, x_hbm,
                         poff_hbm, cnt_hbm, pos_hbm, wpad_hbm, xpad_hbm,
                         idx_t, pos_t, hist_v, lbase_v, cntr_v, thist_v,
                         ghist_v, tbase_v, poff_v, cnts_v,
                         pos_all_v, w_all_v, wpad_v, pos_d, rows_v,
                         hist_sh, pos_sh, sem):
    c = lax.axis_index("c")
    s = lax.axis_index("s")
    lanes = lax.iota(jnp.int32, 16)
    ones = jnp.ones((16,), jnp.int32)
    zeros = jnp.zeros((16,), jnp.int32)

    pltpu.sync_copy(idx_hbm.at[pl.ds(s * 128, 128)], idx_t)

    wid = s * 2 + c
    dbase = wid * _TPW
    xcp = pltpu.async_copy(x_hbm.at[pl.ds(dbase, _TPW)], rows_v, sem)

    def zbody(i, carry):
        hist_v[pl.ds(i * 16, 16)] = zeros
        cntr_v[pl.ds(i * 16, 16)] = zeros
        return carry

    lax.fori_loop(0, 64, zbody, 0)

    for j in range(8):
        e = idx_t[pl.ds(j * 16, 16)]
        plsc.addupdate_scatter(hist_v, [lanes * 64 + e], ones)

    for ch in range(4):
        run = zeros
        for l in range(16):
            lbase_v[pl.ds(l * 64 + ch * 16, 16)] = run
            run = run + hist_v[pl.ds(l * 64 + ch * 16, 16)]
        thist_v[pl.ds(ch * 16, 16)] = run

    pltpu.sync_copy(thist_v, hist_sh.at[pl.ds(s * 64, 64)])
    plsc.subcore_barrier()
    pltpu.sync_copy(hist_sh, ghist_v)
    carry = zeros
    for ch in range(4):
        tbase = zeros
        total = zeros
        for t in range(16):
            g = ghist_v[pl.ds(t * 64 + ch * 16, 16)]
            tbase = tbase + jnp.where(jnp.broadcast_to(t < s, (16,)), g, zeros)
            total = total + g
        tbase_v[pl.ds(ch * 16, 16)] = tbase
        cnts_v[pl.ds(ch * 16, 16)] = total
        pc = ((total + 7) >> 3) << 3
        incl = plsc.cumsum(pc)
        poff_v[pl.ds(ch * 16, 16)] = incl - pc + carry
        carry = carry + jnp.broadcast_to(jnp.sum(pc), (16,))

    for j in range(8):
        e = idx_t[pl.ds(j * 16, 16)]
        stripe = lanes * 64 + e
        base = plsc.load_gather(poff_v, [e])
        tb = plsc.load_gather(tbase_v, [e])
        lb = plsc.load_gather(lbase_v, [stripe])
        cr = plsc.load_gather(cntr_v, [stripe])
        p = base + tb + lb + cr
        plsc.store_scatter(cntr_v, [stripe], cr + ones)
        pos_t[pl.ds(j * 16, 16)] = p

    pltpu.sync_copy(pos_t, pos_sh.at[pl.ds(s * 128, 128)])

    @pl.when(c == 0)
    def _():
        pltpu.sync_copy(pos_t, pos_hbm.at[pl.ds(s * 128, 128)])

    @pl.when((c == 0) & (s == 0))
    def _():
        pltpu.sync_copy(poff_v, poff_hbm)
        pltpu.sync_copy(cnts_v, cnt_hbm)

    plsc.subcore_barrier()

    @pl.when((c == 0) & (s == 0))
    def _():
        pltpu.sync_copy(pos_sh, pos_all_v)
        pltpu.sync_copy(w_hbm, w_all_v)

        def wbody(j, carry):
            wv = w_all_v[pl.ds(j * 16, 16)]
            p = pos_all_v[pl.ds(j * 16, 16)]
            plsc.store_scatter(wpad_v, [p], wv)
            return carry

        lax.fori_loop(0, 128, wbody, 0)
        pltpu.sync_copy(wpad_v, wpad_hbm)

    pltpu.sync_copy(pos_sh.at[pl.ds(dbase, _TPW)], pos_d)
    xcp.wait()
    pltpu.async_copy(rows_v, xpad_hbm.at[pos_d], sem).wait()


def _route_dispatch(idx, w, x):
    return pl.kernel(
        _route_dispatch_body,
        out_type=[
            jax.ShapeDtypeStruct((_E,), jnp.int32),
            jax.ShapeDtypeStruct((_E,), jnp.int32),
            jax.ShapeDtypeStruct((_T,), jnp.int32),
            jax.ShapeDtypeStruct((_N_ROWS,), jnp.float32),
            jax.ShapeDtypeStruct((_N_ROWS, _D), jnp.float32),
        ],
        mesh=plsc.VectorSubcoreMesh(**_MESH),
        compiler_params=pltpu.CompilerParams(needs_layout_passes=False),
        scratch_types=[
            pltpu.VMEM((128,), jnp.int32),
            pltpu.VMEM((128,), jnp.int32),
            pltpu.VMEM((1024,), jnp.int32),
            pltpu.VMEM((1024,), jnp.int32),
            pltpu.VMEM((1024,), jnp.int32),
            pltpu.VMEM((_E,), jnp.int32),
            pltpu.VMEM((1024,), jnp.int32),
            pltpu.VMEM((_E,), jnp.int32),
            pltpu.VMEM((_E,), jnp.int32),
            pltpu.VMEM((_E,), jnp.int32),
            pltpu.VMEM((_T,), jnp.int32),
            pltpu.VMEM((_T,), jnp.float32),
            pltpu.VMEM((_N_ROWS,), jnp.float32),
            pltpu.VMEM((_TPW,), jnp.int32),
            pltpu.VMEM((_TPW, _D), jnp.float32),
            pltpu.VMEM_SHARED((1024,), jnp.int32),
            pltpu.VMEM_SHARED((_T,), jnp.int32),
            pltpu.SemaphoreType.DMA,
        ],
    )(idx, w, x)


def _gmm_body(poff_ref, cnt_ref, x_ref, w_ref, fc1_ref, fc2_ref, out_ref, *,
              bt, d_half, epb):
    g = pl.program_id(0)
    for k in range(epb):
        e = g * epb + k
        start_e = poff_ref[e]
        nt = (cnt_ref[e] + bt - 1) // bt

        def body(i, carry, k=k, start_e=start_e):
            start = pl.multiple_of(start_e + i * bt, 8)
            rows = x_ref[pl.ds(start, bt), :]
            y = lax.dot_general(rows, fc1_ref[k], (((1,), (1,)), ((), ())),
                                preferred_element_type=jnp.float32)
            y1 = y[:, :d_half]
            gate = y[:, d_half:]
            h = y1 * (gate * jax.nn.sigmoid(gate))
            yo = lax.dot_general(h, fc2_ref[k], (((1,), (1,)), ((), ())),
                                 preferred_element_type=jnp.float32)
            out_ref[pl.ds(start, bt), :] = yo * w_ref[pl.ds(start, bt), :]
            return carry

        lax.fori_loop(0, nt, body, 0)


def _grouped_mlp(poff, counts, x_pad, w_pad, fc1_weights, fc2_weights, *,
                 bt=_BT, epb=4):
    n_rows, d_model = x_pad.shape
    n_experts, d_ff2, _ = fc1_weights.shape
    d_half = d_ff2 // 2
    grid_spec = pltpu.PrefetchScalarGridSpec(
        num_scalar_prefetch=2,
        grid=(n_experts // epb,),
        in_specs=[
            pl.BlockSpec((n_rows, d_model), lambda g, poff, cnt: (0, 0)),
            pl.BlockSpec((n_rows, 1), lambda g, poff, cnt: (0, 0)),
            pl.BlockSpec((epb, d_ff2, d_model), lambda g, poff, cnt: (g, 0, 0)),
            pl.BlockSpec((epb, d_model, d_half), lambda g, poff, cnt: (g, 0, 0)),
        ],
        out_specs=pl.BlockSpec((n_rows, d_model), lambda g, poff, cnt: (0, 0)),
    )
    return pl.pallas_call(
        functools.partial(_gmm_body, bt=bt, d_half=d_half, epb=epb),
        grid_spec=grid_spec,
        out_shape=jax.ShapeDtypeStruct((n_rows, d_model), jnp.float32),
    )(poff, counts, x_pad, w_pad, fc1_weights, fc2_weights)


def _combine_body(outpad_hbm, pos_hbm, out_hbm, pos_v, rows_a, rows_b,
                  sem_a, sem_b):
    wid = lax.axis_index("s") * 2 + lax.axis_index("c")
    base = wid * _TPW
    half = _TPW // 2
    pltpu.sync_copy(pos_hbm.at[pl.ds(base, _TPW)], pos_v)
    ca = pltpu.async_copy(outpad_hbm.at[pos_v.at[pl.ds(0, half)]], rows_a,
                          sem_a)
    cb = pltpu.async_copy(outpad_hbm.at[pos_v.at[pl.ds(half, half)]], rows_b,
                          sem_b)
    ca.wait()
    pltpu.sync_copy(rows_a, out_hbm.at[pl.ds(base, half)])
    cb.wait()
    pltpu.sync_copy(rows_b, out_hbm.at[pl.ds(base + half, half)])


def _combine(out_pad, pos):
    return pl.kernel(
        _combine_body,
        out_type=jax.ShapeDtypeStruct((_T, _D), jnp.float32),
        mesh=plsc.VectorSubcoreMesh(**_MESH),
        scratch_types=[
            pltpu.VMEM((_TPW,), jnp.int32),
            pltpu.VMEM((_TPW // 2, _D), jnp.float32),
            pltpu.VMEM((_TPW // 2, _D), jnp.float32),
            pltpu.SemaphoreType.DMA,
            pltpu.SemaphoreType.DMA,
        ],
    )(out_pad, pos)


def kernel(x, weights, indices, fc1_weights, fc2_weights):
    idx = indices.reshape(-1).astype(jnp.int32)
    w = weights.reshape(-1)
    poff, counts, pos, w_pad, x_pad = _route_dispatch(idx, w, x)
    out_pad = _grouped_mlp(poff, counts, x_pad, w_pad.reshape(-1, 1),
                           fc1_weights, fc2_weights)
    return _combine(out_pad, pos)
